# baseline (speedup 1.0000x reference)
import jax
import jax.numpy as jnp
from jax.experimental import pallas as pl
from jax.experimental.pallas import tpu as pltpu

G = 16


def _copy_body(x_ref, o_ref):
    o_ref[...] = x_ref[...]


def _gcn(x, edge_index, W, b):
    n = x.shape[0]
    src = edge_index[0]
    dst = edge_index[1]
    loop = jnp.arange(n, dtype=src.dtype)
    src = jnp.concatenate([src, loop])
    dst = jnp.concatenate([dst, loop])
    deg = jnp.zeros((n,), dtype=x.dtype).at[dst].add(1.0)
    dinv = jnp.where(deg > 0, jax.lax.rsqrt(deg), 0.0)
    norm = dinv[src] * dinv[dst]
    xw = x @ W
    msg = jnp.take(xw, src, axis=0) * norm[:, None]
    agg = jax.ops.segment_sum(msg, dst, num_segments=n)
    return agg + b


def kernel(x, edge_index, batch, W1, b1, W2, b2, Wc, bc):
    h = jax.nn.relu(_gcn(x, edge_index, W1, b1))
    h = jax.nn.relu(_gcn(h, edge_index, W2, b2))
    sums = jax.ops.segment_sum(h, batch, num_segments=G)
    counts = jax.ops.segment_sum(jnp.ones((x.shape[0],), x.dtype), batch, num_segments=G)
    pooled = sums / jnp.maximum(counts, 1.0)[:, None]
    logits = pooled @ Wc + bc
    return pl.pallas_call(_copy_body,
                          out_shape=jax.ShapeDtypeStruct(logits.shape, logits.dtype))(logits)


# speedup vs baseline: 9.4697x; 9.4697x over previous
import functools

import jax
import jax.numpy as jnp
from jax import lax
from jax.experimental import pallas as pl
from jax.experimental.pallas import tpu as pltpu
from jax.experimental.pallas import tpu_sc as plsc

N = 10000
D = 128
E = 320000
G = 16
C = 4

NP = 10240
NC = 2
NS = 16
NW = NC * NS
CHUNK = 128
CPT = 80
GRP = CPT * CHUNK // 16
EP = NW * CPT * CHUNK
RPT = NP // NS

BLK = 1024
NBLK = NP // BLK

_mesh = plsc.VectorSubcoreMesh(core_axis_name="c", subcore_axis_name="s")


def _fill_rows(ref, rows, width, value):
    def body(i, _):
        for j in range(width // 16):
            ref[i, pl.ds(j * 16, 16)] = jnp.full((16,), value, jnp.float32)
        return 0
    lax.fori_loop(0, rows, body, 0)


@functools.partial(
    pl.kernel,
    out_type=jax.ShapeDtypeStruct((NC, NP, 16), jnp.float32),
    mesh=_mesh,
    scratch_types=[
        pltpu.VMEM((GRP, 16), jnp.int32),
        pltpu.VMEM((16, 16), jnp.float32),
        pltpu.VMEM((16, 16), jnp.float32),
        pltpu.VMEM_SHARED((NP, 16), jnp.float32),
        pltpu.SemaphoreType.DMA,
    ],
)
def _deg_kernel(dstw_hbm, out_hbm, dst_v, ones_v, stage_v, deg_sh, sem):
    c = lax.axis_index("c")
    s = lax.axis_index("s")
    w = c * NS + s
    iota = lax.broadcasted_iota(jnp.int32, (16,), 0)
    _fill_rows(ones_v, 16, 16, 1.0)
    _fill_rows(stage_v, 16, 16, 0.0)

    def z(k, _):
        pltpu.sync_copy(stage_v, deg_sh.at[s * RPT + k * 16 + iota])
        return 0
    lax.fori_loop(0, RPT // 16, z, 0)
    plsc.subcore_barrier()

    pltpu.sync_copy(dstw_hbm.at[w], dst_v)

    def step(g, _):
        pltpu.sync_copy(ones_v, deg_sh.at[dst_v[g]], add=True)
        return 0
    lax.fori_loop(0, GRP, step, 0)
    plsc.subcore_barrier()

    def dr(k, _):
        pltpu.async_copy(deg_sh.at[s * RPT + k * 16 + iota], stage_v, sem).wait()
        pltpu.sync_copy(stage_v, out_hbm.at[c, pl.ds(s * RPT + k * 16, 16)])
        return 0
    lax.fori_loop(0, RPT // 16, dr, 0)


@functools.partial(
    pl.kernel,
    out_type=jax.ShapeDtypeStruct((NC, NP, D), jnp.float32),
    mesh=_mesh,
    scratch_types=[
        pltpu.VMEM((CHUNK,), jnp.int32),
        pltpu.VMEM((CHUNK,), jnp.int32),
        pltpu.VMEM((CHUNK,), jnp.int32),
        pltpu.VMEM((CHUNK,), jnp.int32),
        pltpu.VMEM((CHUNK, D), jnp.float32),
        pltpu.VMEM((CHUNK, D), jnp.float32),
        pltpu.VMEM((16, D), jnp.float32),
        pltpu.VMEM_SHARED((NP, D), jnp.float32),
        pltpu.SemaphoreType.DMA,
        pltpu.SemaphoreType.DMA,
        pltpu.SemaphoreType.DMA,
    ],
)
def _scatter_kernel(y_hbm, srcw_hbm, dstw_hbm, out_hbm,
                    ib0, ib1, db0, db1, gb0, gb1, stage_v, acc_sh,
                    sem0, sem1, semd):
    c = lax.axis_index("c")
    s = lax.axis_index("s")
    w = c * NS + s
    iota = lax.broadcasted_iota(jnp.int32, (16,), 0)
    _fill_rows(stage_v, 16, D, 0.0)

    def z(k, _):
        pltpu.sync_copy(stage_v, acc_sh.at[s * RPT + k * 16 + iota])
        return 0
    lax.fori_loop(0, RPT // 16, z, 0)
    plsc.subcore_barrier()

    def stage_and_fire(chunk, ib, db, gb, sem):
        pltpu.sync_copy(srcw_hbm.at[w, chunk], ib)
        pltpu.sync_copy(dstw_hbm.at[w, chunk], db)
        pltpu.async_copy(y_hbm.at[ib], gb, sem)

    def scatters(db, gb):
        for k in range(CHUNK // 16):
            rows = db[pl.ds(k * 16, 16)]
            pltpu.sync_copy(gb.at[pl.ds(k * 16, 16)], acc_sh.at[rows], add=True)

    stage_and_fire(0, ib0, db0, gb0, sem0)
    stage_and_fire(1, ib1, db1, gb1, sem1)

    def pair(t, _):
        a = 2 * t
        pltpu.make_async_copy(y_hbm.at[ib0], gb0, sem0).wait()
        scatters(db0, gb0)

        @pl.when(a + 2 < CPT)
        def _():
            stage_and_fire(a + 2, ib0, db0, gb0, sem0)

        pltpu.make_async_copy(y_hbm.at[ib1], gb1, sem1).wait()
        scatters(db1, gb1)

        @pl.when(a + 3 < CPT)
        def _():
            stage_and_fire(a + 3, ib1, db1, gb1, sem1)
        return 0
    lax.fori_loop(0, CPT // 2, pair, 0)
    plsc.subcore_barrier()

    def dr(k, _):
        pltpu.async_copy(acc_sh.at[s * RPT + k * 16 + iota], stage_v, semd).wait()
        pltpu.sync_copy(stage_v, out_hbm.at[c, pl.ds(s * RPT + k * 16, 16)])
        return 0
    lax.fori_loop(0, RPT // 16, dr, 0)


def _dinv_of(degp_blk):
    deg = degp_blk[0, :, 0:1] + degp_blk[1, :, 0:1] + 1.0
    return lax.rsqrt(deg)


def _y1_body(x_ref, w_ref, degp_ref, y_ref):
    xw = jnp.dot(x_ref[...], w_ref[...], preferred_element_type=jnp.float32)
    y_ref[...] = xw * _dinv_of(degp_ref)


def _mid_body(accp_ref, y1_ref, degp_ref, w2_ref, b1_ref, y2_ref):
    dinv = _dinv_of(degp_ref)
    h = accp_ref[0] + accp_ref[1] + y1_ref[...]
    h = jnp.maximum(h * dinv + b1_ref[...], 0.0)
    y2_ref[...] = jnp.dot(h, w2_ref[...], preferred_element_type=jnp.float32) * dinv


def _fin_body(accp_ref, y2_ref, degp_ref, b2_ref, batch_ref, wc_ref, bc_ref,
              out_ref, sums, cnts):
    i = pl.program_id(0)

    @pl.when(i == 0)
    def _():
        sums[...] = jnp.zeros_like(sums)
        cnts[...] = jnp.zeros_like(cnts)

    dinv = _dinv_of(degp_ref)
    h = accp_ref[0] + accp_ref[1] + y2_ref[...]
    h = jnp.maximum(h * dinv + b2_ref[...], 0.0)
    b = batch_ref[0]
    gi = lax.broadcasted_iota(jnp.int32, (G, BLK), 0)
    onehot_t = (b == gi).astype(jnp.float32)
    sums[...] += jnp.dot(onehot_t, h, preferred_element_type=jnp.float32)
    cnts[...] += jnp.sum(onehot_t, axis=1, keepdims=True)

    @pl.when(i == NBLK - 1)
    def _():
        pooled = sums[...] / jnp.maximum(cnts[...], 1.0)
        out_ref[...] = (jnp.dot(pooled, wc_ref[...],
                                preferred_element_type=jnp.float32) + bc_ref[...])


def _row_spec(): return pl.BlockSpec((BLK, D), lambda i: (i, 0))
def _degp_spec(): return pl.BlockSpec((NC, BLK, 16), lambda i: (0, i, 0))
def _full_spec(shape): return pl.BlockSpec(shape, lambda i: tuple(0 for _ in shape))


_y1_call = pl.pallas_call(
    _y1_body,
    grid=(NBLK,),
    in_specs=[_row_spec(), _full_spec((D, D)), _degp_spec()],
    out_specs=_row_spec(),
    out_shape=jax.ShapeDtypeStruct((NP, D), jnp.float32),
)

_mid_call = pl.pallas_call(
    _mid_body,
    grid=(NBLK,),
    in_specs=[pl.BlockSpec((NC, BLK, D), lambda i: (0, i, 0)), _row_spec(),
              _degp_spec(), _full_spec((D, D)), _full_spec((1, D))],
    out_specs=_row_spec(),
    out_shape=jax.ShapeDtypeStruct((NP, D), jnp.float32),
)

_fin_call = pl.pallas_call(
    _fin_body,
    grid=(NBLK,),
    in_specs=[pl.BlockSpec((NC, BLK, D), lambda i: (0, i, 0)), _row_spec(),
              _degp_spec(), _full_spec((1, D)),
              pl.BlockSpec((1, 1, BLK), lambda i: (i, 0, 0)),
              _full_spec((D, D)), _full_spec((1, D))],
    out_specs=_full_spec((G, D)),
    out_shape=jax.ShapeDtypeStruct((G, D), jnp.float32),
    scratch_shapes=[pltpu.VMEM((G, D), jnp.float32),
                    pltpu.VMEM((G, D), jnp.float32)],
)


def kernel(x, edge_index, batch, W1, b1, W2, b2, Wc, bc):
    x_p = jnp.pad(x, ((0, NP - N), (0, 0)))
    src = jnp.pad(edge_index[0], (0, EP - E), constant_values=NP - 1)
    dst = jnp.pad(edge_index[1], (0, EP - E), constant_values=NP - 1)
    srcw = src.reshape(NW, CPT, CHUNK)
    dstw = dst.reshape(NW, CPT, CHUNK)
    dstg = dst.reshape(NW, GRP, 16)
    batch3 = jnp.pad(batch, (0, NP - N), constant_values=G).reshape(NBLK, 1, BLK)
    wc_p = jnp.pad(Wc, ((0, 0), (0, D - C)))
    bc_p = jnp.pad(bc, (0, D - C)).reshape(1, D)

    degp = _deg_kernel(dstg)
    y1 = _y1_call(x_p, W1, degp)
    acc1 = _scatter_kernel(y1, srcw, dstw)
    y2 = _mid_call(acc1, y1, degp, W2, b1.reshape(1, D))
    acc2 = _scatter_kernel(y2, srcw, dstw)
    outp = _fin_call(acc2, y2, degp, b2.reshape(1, D), batch3, wc_p, bc_p)
    return outp[:, :C]


# speedup vs baseline: 9.8438x; 1.0395x over previous
import functools

import jax
import jax.numpy as jnp
from jax import lax
from jax.experimental import pallas as pl
from jax.experimental.pallas import tpu as pltpu
from jax.experimental.pallas import tpu_sc as plsc

N = 10000
D = 128
E = 320000
G = 16
C = 4

NP = 10240
NC = 2
NS = 16
NW = NC * NS
CHUNK = 128
CPT = 80
CPT0 = 116
CPT1 = 44
GRP = CPT * CHUNK // 16
EP = NW * CPT * CHUNK
RPT = NP // NS

BLK = 1024
NBLK = NP // BLK

_mesh = plsc.VectorSubcoreMesh(core_axis_name="c", subcore_axis_name="s")


def _fill_rows(ref, rows, width, value):
    def body(i, _):
        for j in range(width // 16):
            ref[i, pl.ds(j * 16, 16)] = jnp.full((16,), value, jnp.float32)
        return 0
    lax.fori_loop(0, rows, body, 0)


@functools.partial(
    pl.kernel,
    out_type=jax.ShapeDtypeStruct((NC, NP, 16), jnp.float32),
    mesh=_mesh,
    scratch_types=[
        pltpu.VMEM((GRP, 16), jnp.int32),
        pltpu.VMEM((16, 16), jnp.float32),
        pltpu.VMEM((16, 16), jnp.float32),
        pltpu.VMEM_SHARED((NP, 16), jnp.float32),
        pltpu.SemaphoreType.DMA,
    ],
)
def _deg_kernel(dstw_hbm, out_hbm, dst_v, ones_v, stage_v, deg_sh, sem):
    c = lax.axis_index("c")
    s = lax.axis_index("s")
    w = c * NS + s
    iota = lax.broadcasted_iota(jnp.int32, (16,), 0)
    _fill_rows(ones_v, 16, 16, 1.0)
    _fill_rows(stage_v, 16, 16, 0.0)

    def z(k, _):
        pltpu.sync_copy(stage_v, deg_sh.at[s * RPT + k * 16 + iota])
        return 0
    lax.fori_loop(0, RPT // 16, z, 0)
    plsc.subcore_barrier()

    pltpu.sync_copy(dstw_hbm.at[w], dst_v)

    def step(g, _):
        pltpu.sync_copy(ones_v, deg_sh.at[dst_v[g]], add=True)
        return 0
    lax.fori_loop(0, GRP, step, 0)
    plsc.subcore_barrier()

    def dr(k, _):
        pltpu.async_copy(deg_sh.at[s * RPT + k * 16 + iota], stage_v, sem).wait()
        pltpu.sync_copy(stage_v, out_hbm.at[c, pl.ds(s * RPT + k * 16, 16)])
        return 0
    lax.fori_loop(0, RPT // 16, dr, 0)


@functools.partial(
    pl.kernel,
    out_type=jax.ShapeDtypeStruct((NC, NP, D), jnp.float32),
    mesh=_mesh,
    scratch_types=[
        pltpu.VMEM((CHUNK,), jnp.int32),
        pltpu.VMEM((CHUNK,), jnp.int32),
        pltpu.VMEM((CHUNK,), jnp.int32),
        pltpu.VMEM((CHUNK,), jnp.int32),
        pltpu.VMEM((CHUNK, D), jnp.float32),
        pltpu.VMEM((CHUNK, D), jnp.float32),
        pltpu.VMEM((16, D), jnp.float32),
        pltpu.VMEM_SHARED((NP, D), jnp.float32),
        pltpu.SemaphoreType.DMA,
        pltpu.SemaphoreType.DMA,
        pltpu.SemaphoreType.DMA,
    ],
)
def _scatter_kernel(y_hbm, src0_hbm, dst0_hbm, src1_hbm, dst1_hbm, out_hbm,
                    ib0, ib1, db0, db1, gb0, gb1, stage_v, acc_sh,
                    sem0, sem1, semd):
    c = lax.axis_index("c")
    s = lax.axis_index("s")
    iota = lax.broadcasted_iota(jnp.int32, (16,), 0)
    _fill_rows(stage_v, 16, D, 0.0)

    def z(k, _):
        pltpu.sync_copy(stage_v, acc_sh.at[s * RPT + k * 16 + iota])
        return 0
    lax.fori_loop(0, RPT // 16, z, 0)
    plsc.subcore_barrier()

    def run_pipeline(srcw_hbm, dstw_hbm, cpt):
        def stage_and_fire(chunk, ib, db, gb, sem):
            pltpu.sync_copy(srcw_hbm.at[s, chunk], ib)
            pltpu.sync_copy(dstw_hbm.at[s, chunk], db)
            pltpu.async_copy(y_hbm.at[ib], gb, sem)

        def scatters(db, gb):
            for k in range(CHUNK // 16):
                rows = db[pl.ds(k * 16, 16)]
                pltpu.sync_copy(gb.at[pl.ds(k * 16, 16)], acc_sh.at[rows],
                                add=True)

        stage_and_fire(0, ib0, db0, gb0, sem0)
        stage_and_fire(1, ib1, db1, gb1, sem1)

        def pair(t, _):
            a = 2 * t
            pltpu.make_async_copy(y_hbm.at[ib0], gb0, sem0).wait()
            scatters(db0, gb0)

            @pl.when(a + 2 < cpt)
            def _():
                stage_and_fire(a + 2, ib0, db0, gb0, sem0)

            pltpu.make_async_copy(y_hbm.at[ib1], gb1, sem1).wait()
            scatters(db1, gb1)

            @pl.when(a + 3 < cpt)
            def _():
                stage_and_fire(a + 3, ib1, db1, gb1, sem1)
            return 0
        lax.fori_loop(0, cpt // 2, pair, 0)

    @pl.when(c == 0)
    def _():
        run_pipeline(src0_hbm, dst0_hbm, CPT0)

    @pl.when(c == 1)
    def _():
        run_pipeline(src1_hbm, dst1_hbm, CPT1)

    plsc.subcore_barrier()

    def dr(k, _):
        pltpu.async_copy(acc_sh.at[s * RPT + k * 16 + iota], stage_v, semd).wait()
        pltpu.sync_copy(stage_v, out_hbm.at[c, pl.ds(s * RPT + k * 16, 16)])
        return 0
    lax.fori_loop(0, RPT // 16, dr, 0)


def _dinv_of(degp_blk):
    deg = degp_blk[0, :, 0:1] + degp_blk[1, :, 0:1] + 1.0
    return lax.rsqrt(deg)


def _y1_body(x_ref, w_ref, degp_ref, y_ref):
    xw = jnp.dot(x_ref[...], w_ref[...], preferred_element_type=jnp.float32)
    y_ref[...] = xw * _dinv_of(degp_ref)


def _mid_body(accp_ref, y1_ref, degp_ref, w2_ref, b1_ref, y2_ref):
    dinv = _dinv_of(degp_ref)
    h = accp_ref[0] + accp_ref[1] + y1_ref[...]
    h = jnp.maximum(h * dinv + b1_ref[...], 0.0)
    y2_ref[...] = jnp.dot(h, w2_ref[...], preferred_element_type=jnp.float32) * dinv


def _fin_body(accp_ref, y2_ref, degp_ref, b2_ref, batch_ref, wc_ref, bc_ref,
              out_ref, sums, cnts):
    i = pl.program_id(0)

    @pl.when(i == 0)
    def _():
        sums[...] = jnp.zeros_like(sums)
        cnts[...] = jnp.zeros_like(cnts)

    dinv = _dinv_of(degp_ref)
    h = accp_ref[0] + accp_ref[1] + y2_ref[...]
    h = jnp.maximum(h * dinv + b2_ref[...], 0.0)
    b = batch_ref[0]
    gi = lax.broadcasted_iota(jnp.int32, (G, BLK), 0)
    onehot_t = (b == gi).astype(jnp.float32)
    sums[...] += jnp.dot(onehot_t, h, preferred_element_type=jnp.float32)
    cnts[...] += jnp.sum(onehot_t, axis=1, keepdims=True)

    @pl.when(i == NBLK - 1)
    def _():
        pooled = sums[...] / jnp.maximum(cnts[...], 1.0)
        out_ref[...] = (jnp.dot(pooled, wc_ref[...],
                                preferred_element_type=jnp.float32) + bc_ref[...])


def _row_spec(): return pl.BlockSpec((BLK, D), lambda i: (i, 0))
def _degp_spec(): return pl.BlockSpec((NC, BLK, 16), lambda i: (0, i, 0))
def _full_spec(shape): return pl.BlockSpec(shape, lambda i: tuple(0 for _ in shape))


_y1_call = pl.pallas_call(
    _y1_body,
    grid=(NBLK,),
    in_specs=[_row_spec(), _full_spec((D, D)), _degp_spec()],
    out_specs=_row_spec(),
    out_shape=jax.ShapeDtypeStruct((NP, D), jnp.float32),
)

_mid_call = pl.pallas_call(
    _mid_body,
    grid=(NBLK,),
    in_specs=[pl.BlockSpec((NC, BLK, D), lambda i: (0, i, 0)), _row_spec(),
              _degp_spec(), _full_spec((D, D)), _full_spec((1, D))],
    out_specs=_row_spec(),
    out_shape=jax.ShapeDtypeStruct((NP, D), jnp.float32),
)

_fin_call = pl.pallas_call(
    _fin_body,
    grid=(NBLK,),
    in_specs=[pl.BlockSpec((NC, BLK, D), lambda i: (0, i, 0)), _row_spec(),
              _degp_spec(), _full_spec((1, D)),
              pl.BlockSpec((1, 1, BLK), lambda i: (i, 0, 0)),
              _full_spec((D, D)), _full_spec((1, D))],
    out_specs=_full_spec((G, D)),
    out_shape=jax.ShapeDtypeStruct((G, D), jnp.float32),
    scratch_shapes=[pltpu.VMEM((G, D), jnp.float32),
                    pltpu.VMEM((G, D), jnp.float32)],
)


def kernel(x, edge_index, batch, W1, b1, W2, b2, Wc, bc):
    x_p = jnp.pad(x, ((0, NP - N), (0, 0)))
    src = jnp.pad(edge_index[0], (0, EP - E), constant_values=NP - 1)
    dst = jnp.pad(edge_index[1], (0, EP - E), constant_values=NP - 1)
    e0 = NS * CPT0 * CHUNK
    src0 = src[:e0].reshape(NS, CPT0, CHUNK)
    dst0 = dst[:e0].reshape(NS, CPT0, CHUNK)
    src1 = src[e0:].reshape(NS, CPT1, CHUNK)
    dst1 = dst[e0:].reshape(NS, CPT1, CHUNK)
    dstg = dst.reshape(NW, GRP, 16)
    batch3 = jnp.pad(batch, (0, NP - N), constant_values=G).reshape(NBLK, 1, BLK)
    wc_p = jnp.pad(Wc, ((0, 0), (0, D - C)))
    bc_p = jnp.pad(bc, (0, D - C)).reshape(1, D)

    degp = _deg_kernel(dstg)
    y1 = _y1_call(x_p, W1, degp)
    acc1 = _scatter_kernel(y1, src0, dst0, src1, dst1)
    y2 = _mid_call(acc1, y1, degp, W2, b1.reshape(1, D))
    acc2 = _scatter_kernel(y2, src0, dst0, src1, dst1)
    outp = _fin_call(acc2, y2, degp, b2.reshape(1, D), batch3, wc_p, bc_p)
    return outp[:, :C]


# speedup vs baseline: 20.5659x; 2.0892x over previous
import functools

import jax
import jax.numpy as jnp
from jax import lax
from jax.experimental import pallas as pl
from jax.experimental.pallas import tpu as pltpu
from jax.experimental.pallas import tpu_sc as plsc

N = 10000
D = 128
E = 320000
G = 16
C = 4

NP = 10240
NC = 2
NS = 16
NW = NC * NS
CHUNK = 128
CPT = 80
GRP = CPT * CHUNK // 16
EP = NW * CPT * CHUNK
RPT = NP // NS

BLK = 1024
NBLK = NP // BLK

_mesh = plsc.VectorSubcoreMesh(core_axis_name="c", subcore_axis_name="s")


def _fill_rows(ref, rows, width, value):
    def body(i, _):
        for j in range(width // 16):
            ref[i, pl.ds(j * 16, 16)] = jnp.full((16,), value, jnp.float32)
        return 0
    lax.fori_loop(0, rows, body, 0)


@functools.partial(
    pl.kernel,
    out_type=jax.ShapeDtypeStruct((NC, NP, 16), jnp.float32),
    mesh=_mesh,
    scratch_types=[
        pltpu.VMEM((GRP, 16), jnp.int32),
        pltpu.VMEM((16, 16), jnp.float32),
        pltpu.VMEM((16, 16), jnp.float32),
        pltpu.VMEM_SHARED((NP, 16), jnp.float32),
        pltpu.SemaphoreType.DMA,
    ],
)
def _deg_kernel(dstw_hbm, out_hbm, dst_v, ones_v, stage_v, deg_sh, sem):
    c = lax.axis_index("c")
    s = lax.axis_index("s")
    w = c * NS + s
    iota = lax.broadcasted_iota(jnp.int32, (16,), 0)
    _fill_rows(ones_v, 16, 16, 1.0)
    _fill_rows(stage_v, 16, 16, 0.0)

    def z(k, _):
        pltpu.sync_copy(stage_v, deg_sh.at[s * RPT + k * 16 + iota])
        return 0
    lax.fori_loop(0, RPT // 16, z, 0)
    plsc.subcore_barrier()

    pltpu.sync_copy(dstw_hbm.at[w], dst_v)

    def step(g, _):
        pltpu.sync_copy(ones_v, deg_sh.at[dst_v[g]], add=True)
        return 0
    lax.fori_loop(0, GRP, step, 0)
    plsc.subcore_barrier()

    def dr(k, _):
        pltpu.async_copy(deg_sh.at[s * RPT + k * 16 + iota], stage_v, sem).wait()
        pltpu.sync_copy(stage_v, out_hbm.at[c, pl.ds(s * RPT + k * 16, 16)])
        return 0
    lax.fori_loop(0, RPT // 16, dr, 0)


@functools.partial(
    pl.kernel,
    out_type=jax.ShapeDtypeStruct((NC, NP, D), jnp.float32),
    mesh=_mesh,
    scratch_types=[
        pltpu.VMEM((CHUNK,), jnp.int32),
        pltpu.VMEM((CHUNK,), jnp.int32),
        pltpu.VMEM((CHUNK,), jnp.int32),
        pltpu.VMEM((CHUNK,), jnp.int32),
        pltpu.VMEM((CHUNK, D), jnp.float32),
        pltpu.VMEM((CHUNK, D), jnp.float32),
        pltpu.VMEM((16, D), jnp.float32),
        pltpu.VMEM_SHARED((NP, D), jnp.float32),
        pltpu.SemaphoreType.DMA,
        pltpu.SemaphoreType.DMA,
        pltpu.SemaphoreType.DMA,
    ],
)
def _scatter_kernel(y_hbm, srcw_hbm, dstw_hbm, out_hbm,
                    ib0, ib1, db0, db1, gb0, gb1, stage_v, acc_sh,
                    sem0, sem1, semd):
    c = lax.axis_index("c")
    s = lax.axis_index("s")
    w = c * NS + s
    iota = lax.broadcasted_iota(jnp.int32, (16,), 0)
    _fill_rows(stage_v, 16, D, 0.0)

    def z(k, _):
        pltpu.sync_copy(stage_v, acc_sh.at[s * RPT + k * 16 + iota])
        return 0
    lax.fori_loop(0, RPT // 16, z, 0)
    plsc.subcore_barrier()

    def stage_and_fire(chunk, ib, db, gb, sem):
        pltpu.sync_copy(srcw_hbm.at[w, chunk], ib)
        pltpu.sync_copy(dstw_hbm.at[w, chunk], db)
        pltpu.async_copy(y_hbm.at[ib], gb, sem)

    def scatters(db, gb):
        for k in range(CHUNK // 16):
            rows = db[pl.ds(k * 16, 16)]
            pltpu.sync_copy(gb.at[pl.ds(k * 16, 16)], acc_sh.at[rows],
                            add=True)

    stage_and_fire(0, ib0, db0, gb0, sem0)
    stage_and_fire(1, ib1, db1, gb1, sem1)

    def pair(t, _):
        a = 2 * t
        pltpu.make_async_copy(y_hbm.at[ib0], gb0, sem0).wait()
        scatters(db0, gb0)

        @pl.when(a + 2 < CPT)
        def _():
            stage_and_fire(a + 2, ib0, db0, gb0, sem0)

        pltpu.make_async_copy(y_hbm.at[ib1], gb1, sem1).wait()
        scatters(db1, gb1)

        @pl.when(a + 3 < CPT)
        def _():
            stage_and_fire(a + 3, ib1, db1, gb1, sem1)
        return 0
    lax.fori_loop(0, CPT // 2, pair, 0)
    plsc.subcore_barrier()

    def dr(k, _):
        pltpu.async_copy(acc_sh.at[s * RPT + k * 16 + iota], stage_v, semd).wait()
        pltpu.sync_copy(stage_v, out_hbm.at[c, pl.ds(s * RPT + k * 16, 16)])
        return 0
    lax.fori_loop(0, RPT // 16, dr, 0)


def _dinv_of(degp_blk):
    deg = degp_blk[0, :, 0:1] + degp_blk[1, :, 0:1] + 1.0
    return lax.rsqrt(deg)


def _y1_body(x_ref, w_ref, degp_ref, y_ref):
    xw = jnp.dot(x_ref[...], w_ref[...], preferred_element_type=jnp.float32)
    y_ref[...] = xw * _dinv_of(degp_ref)


def _mid_body(accp_ref, y1_ref, degp_ref, w2_ref, b1_ref, y2_ref):
    dinv = _dinv_of(degp_ref)
    h = accp_ref[0] + accp_ref[1] + y1_ref[...]
    h = jnp.maximum(h * dinv + b1_ref[...], 0.0)
    y2_ref[...] = jnp.dot(h, w2_ref[...], preferred_element_type=jnp.float32) * dinv


def _fin_body(accp_ref, y2_ref, degp_ref, b2_ref, batch_ref, wc_ref, bc_ref,
              out_ref, sums, cnts):
    i = pl.program_id(0)

    @pl.when(i == 0)
    def _():
        sums[...] = jnp.zeros_like(sums)
        cnts[...] = jnp.zeros_like(cnts)

    dinv = _dinv_of(degp_ref)
    h = accp_ref[0] + accp_ref[1] + y2_ref[...]
    h = jnp.maximum(h * dinv + b2_ref[...], 0.0)
    b = batch_ref[0]
    gi = lax.broadcasted_iota(jnp.int32, (G, BLK), 0)
    onehot_t = (b == gi).astype(jnp.float32)
    sums[...] += jnp.dot(onehot_t, h, preferred_element_type=jnp.float32)
    cnts[...] += jnp.sum(onehot_t, axis=1, keepdims=True)

    @pl.when(i == NBLK - 1)
    def _():
        pooled = sums[...] / jnp.maximum(cnts[...], 1.0)
        out_ref[...] = (jnp.dot(pooled, wc_ref[...],
                                preferred_element_type=jnp.float32) + bc_ref[...])


def _row_spec(): return pl.BlockSpec((BLK, D), lambda i: (i, 0))
def _degp_spec(): return pl.BlockSpec((NC, BLK, 16), lambda i: (0, i, 0))
def _full_spec(shape): return pl.BlockSpec(shape, lambda i: tuple(0 for _ in shape))


_y1_call = pl.pallas_call(
    _y1_body,
    grid=(NBLK,),
    in_specs=[_row_spec(), _full_spec((D, D)), _degp_spec()],
    out_specs=_row_spec(),
    out_shape=jax.ShapeDtypeStruct((NP, D), jnp.float32),
)

_mid_call = pl.pallas_call(
    _mid_body,
    grid=(NBLK,),
    in_specs=[pl.BlockSpec((NC, BLK, D), lambda i: (0, i, 0)), _row_spec(),
              _degp_spec(), _full_spec((D, D)), _full_spec((1, D))],
    out_specs=_row_spec(),
    out_shape=jax.ShapeDtypeStruct((NP, D), jnp.float32),
)

_fin_call = pl.pallas_call(
    _fin_body,
    grid=(NBLK,),
    in_specs=[pl.BlockSpec((NC, BLK, D), lambda i: (0, i, 0)), _row_spec(),
              _degp_spec(), _full_spec((1, D)),
              pl.BlockSpec((1, 1, BLK), lambda i: (i, 0, 0)),
              _full_spec((D, D)), _full_spec((1, D))],
    out_specs=_full_spec((G, D)),
    out_shape=jax.ShapeDtypeStruct((G, D), jnp.float32),
    scratch_shapes=[pltpu.VMEM((G, D), jnp.float32),
                    pltpu.VMEM((G, D), jnp.float32)],
)


def kernel(x, edge_index, batch, W1, b1, W2, b2, Wc, bc):
    x_p = jnp.pad(x, ((0, NP - N), (0, 0)))
    pad_src = jnp.arange(EP - E, dtype=jnp.int32) % N
    src = jnp.concatenate([edge_index[0], pad_src])
    dst = jnp.pad(edge_index[1], (0, EP - E), constant_values=NP - 1)
    srcw = src.reshape(NW, CPT, CHUNK)
    dstw = dst.reshape(NW, CPT, CHUNK)
    dstg = dst.reshape(NW, GRP, 16)
    batch3 = jnp.pad(batch, (0, NP - N), constant_values=G).reshape(NBLK, 1, BLK)
    wc_p = jnp.pad(Wc, ((0, 0), (0, D - C)))
    bc_p = jnp.pad(bc, (0, D - C)).reshape(1, D)

    degp = _deg_kernel(dstg)
    y1 = _y1_call(x_p, W1, degp)
    acc1 = _scatter_kernel(y1, srcw, dstw)
    y2 = _mid_call(acc1, y1, degp, W2, b1.reshape(1, D))
    acc2 = _scatter_kernel(y2, srcw, dstw)
    outp = _fin_call(acc2, y2, degp, b2.reshape(1, D), batch3, wc_p, bc_p)
    return outp[:, :C]


# speedup vs baseline: 23.7688x; 1.1557x over previous
import functools

import jax
import jax.numpy as jnp
from jax import lax
from jax.experimental import pallas as pl
from jax.experimental.pallas import tpu as pltpu
from jax.experimental.pallas import tpu_sc as plsc

N = 10000
D = 128
E = 320000
G = 16
C = 4

NP = 10240
NC = 2
NS = 16
NW = NC * NS
CHUNK = 128
CPT = 80
GRP = CPT * CHUNK // 16
EP = NW * CPT * CHUNK
RPT = NP // NS

BLK = 1024
NBLK = NP // BLK

_mesh = plsc.VectorSubcoreMesh(core_axis_name="c", subcore_axis_name="s")


def _fill_rows(ref, rows, width, value):
    def body(i, _):
        for j in range(width // 16):
            ref[i, pl.ds(j * 16, 16)] = jnp.full((16,), value, jnp.float32)
        return 0
    lax.fori_loop(0, rows, body, 0)


@functools.partial(
    pl.kernel,
    out_type=jax.ShapeDtypeStruct((NC, NP, 16), jnp.float32),
    mesh=_mesh,
    scratch_types=[
        pltpu.VMEM((GRP, 16), jnp.int32),
        pltpu.VMEM((16, 16), jnp.float32),
        pltpu.VMEM((16, 16), jnp.float32),
        pltpu.VMEM_SHARED((NP, 16), jnp.float32),
        pltpu.SemaphoreType.DMA,
    ],
)
def _deg_kernel(dstw_hbm, out_hbm, dst_v, ones_v, stage_v, deg_sh, sem):
    c = lax.axis_index("c")
    s = lax.axis_index("s")
    w = c * NS + s
    iota = lax.broadcasted_iota(jnp.int32, (16,), 0)
    _fill_rows(ones_v, 16, 16, 1.0)
    _fill_rows(stage_v, 16, 16, 0.0)

    def z(k, _):
        pltpu.sync_copy(stage_v, deg_sh.at[s * RPT + k * 16 + iota])
        return 0
    lax.fori_loop(0, RPT // 16, z, 0)
    plsc.subcore_barrier()

    pltpu.sync_copy(dstw_hbm.at[w], dst_v)

    def step(g, _):
        pltpu.sync_copy(ones_v, deg_sh.at[dst_v[g]], add=True)
        return 0
    lax.fori_loop(0, GRP, step, 0)
    plsc.subcore_barrier()

    def dr(k, _):
        pltpu.async_copy(deg_sh.at[s * RPT + k * 16 + iota], stage_v, sem).wait()
        pltpu.sync_copy(stage_v, out_hbm.at[c, pl.ds(s * RPT + k * 16, 16)])
        return 0
    lax.fori_loop(0, RPT // 16, dr, 0)


@functools.partial(
    pl.kernel,
    out_type=jax.ShapeDtypeStruct((NC, NP, D), jnp.float32),
    mesh=_mesh,
    scratch_types=[
        pltpu.VMEM((CHUNK,), jnp.int32),
        pltpu.VMEM((CHUNK,), jnp.int32),
        pltpu.VMEM((CHUNK,), jnp.int32),
        pltpu.VMEM((CHUNK,), jnp.int32),
        pltpu.VMEM((CHUNK, D), jnp.float32),
        pltpu.VMEM((CHUNK, D), jnp.float32),
        pltpu.VMEM((16, D), jnp.float32),
        pltpu.VMEM((16, D), jnp.float32),
        pltpu.VMEM_SHARED((NP, D), jnp.float32),
        pltpu.SemaphoreType.DMA,
        pltpu.SemaphoreType.DMA,
        pltpu.SemaphoreType.DMA,
        pltpu.SemaphoreType.DMA,
        pltpu.SemaphoreType.DMA,
    ],
)
def _scatter_kernel(y_hbm, srcw_hbm, dstw_hbm, out_hbm,
                    ib0, ib1, db0, db1, gb0, gb1, stage_v, stage2_v, acc_sh,
                    sem0, sem1, semd0, semd1, semsc):
    c = lax.axis_index("c")
    s = lax.axis_index("s")
    w = c * NS + s
    iota = lax.broadcasted_iota(jnp.int32, (16,), 0)
    _fill_rows(stage_v, 16, D, 0.0)

    def z(k, _):
        pltpu.sync_copy(stage_v, acc_sh.at[s * RPT + k * 16 + iota])
        return 0
    lax.fori_loop(0, RPT // 16, z, 0)
    plsc.subcore_barrier()

    def stage_and_fire(chunk, ib, db, gb, sem):
        pltpu.sync_copy(srcw_hbm.at[w, chunk], ib)
        pltpu.sync_copy(dstw_hbm.at[w, chunk], db)
        pltpu.async_copy(y_hbm.at[ib], gb, sem)

    def scatters(db, gb):
        for k in range(CHUNK // 16):
            rows = db[pl.ds(k * 16, 16)]
            pltpu.async_copy(gb.at[pl.ds(k * 16, 16)], acc_sh.at[rows],
                             semsc, add=True)
        for k in range(CHUNK // 16):
            rows = db[pl.ds(k * 16, 16)]
            pltpu.make_async_copy(gb.at[pl.ds(k * 16, 16)], acc_sh.at[rows],
                                  semsc).wait()

    stage_and_fire(0, ib0, db0, gb0, sem0)
    stage_and_fire(1, ib1, db1, gb1, sem1)

    def pair(t, _):
        a = 2 * t
        pltpu.make_async_copy(y_hbm.at[ib0], gb0, sem0).wait()
        scatters(db0, gb0)

        @pl.when(a + 2 < CPT)
        def _():
            stage_and_fire(a + 2, ib0, db0, gb0, sem0)

        pltpu.make_async_copy(y_hbm.at[ib1], gb1, sem1).wait()
        scatters(db1, gb1)

        @pl.when(a + 3 < CPT)
        def _():
            stage_and_fire(a + 3, ib1, db1, gb1, sem1)
        return 0
    lax.fori_loop(0, CPT // 2, pair, 0)
    plsc.subcore_barrier()

    pltpu.async_copy(acc_sh.at[s * RPT + iota], stage_v, semd0)
    pltpu.async_copy(acc_sh.at[s * RPT + 16 + iota], stage2_v, semd1)

    def drpair(t, _):
        k = 2 * t
        pltpu.make_async_copy(acc_sh.at[s * RPT + iota], stage_v, semd0).wait()
        pltpu.sync_copy(stage_v, out_hbm.at[c, pl.ds(s * RPT + k * 16, 16)])

        @pl.when(k + 2 < RPT // 16)
        def _():
            pltpu.async_copy(acc_sh.at[s * RPT + (k + 2) * 16 + iota],
                             stage_v, semd0)

        pltpu.make_async_copy(acc_sh.at[s * RPT + iota], stage2_v, semd1).wait()
        pltpu.sync_copy(stage2_v, out_hbm.at[c, pl.ds(s * RPT + (k + 1) * 16, 16)])

        @pl.when(k + 3 < RPT // 16)
        def _():
            pltpu.async_copy(acc_sh.at[s * RPT + (k + 3) * 16 + iota],
                             stage2_v, semd1)
        return 0
    lax.fori_loop(0, RPT // 32, drpair, 0)


def _dinv_of(degp_blk):
    deg = degp_blk[0, :, 0:1] + degp_blk[1, :, 0:1] + 1.0
    return lax.rsqrt(deg)


def _y1_body(x_ref, w_ref, degp_ref, y_ref):
    xw = jnp.dot(x_ref[...], w_ref[...], preferred_element_type=jnp.float32)
    y_ref[...] = xw * _dinv_of(degp_ref)


def _mid_body(accp_ref, y1_ref, degp_ref, w2_ref, b1_ref, y2_ref):
    dinv = _dinv_of(degp_ref)
    h = accp_ref[0] + accp_ref[1] + y1_ref[...]
    h = jnp.maximum(h * dinv + b1_ref[...], 0.0)
    y2_ref[...] = jnp.dot(h, w2_ref[...], preferred_element_type=jnp.float32) * dinv


def _fin_body(accp_ref, y2_ref, degp_ref, b2_ref, batch_ref, wc_ref, bc_ref,
              out_ref, sums, cnts):
    i = pl.program_id(0)

    @pl.when(i == 0)
    def _():
        sums[...] = jnp.zeros_like(sums)
        cnts[...] = jnp.zeros_like(cnts)

    dinv = _dinv_of(degp_ref)
    h = accp_ref[0] + accp_ref[1] + y2_ref[...]
    h = jnp.maximum(h * dinv + b2_ref[...], 0.0)
    b = batch_ref[0]
    gi = lax.broadcasted_iota(jnp.int32, (G, BLK), 0)
    onehot_t = (b == gi).astype(jnp.float32)
    sums[...] += jnp.dot(onehot_t, h, preferred_element_type=jnp.float32)
    cnts[...] += jnp.sum(onehot_t, axis=1, keepdims=True)

    @pl.when(i == NBLK - 1)
    def _():
        pooled = sums[...] / jnp.maximum(cnts[...], 1.0)
        out_ref[...] = (jnp.dot(pooled, wc_ref[...],
                                preferred_element_type=jnp.float32) + bc_ref[...])


def _row_spec(): return pl.BlockSpec((BLK, D), lambda i: (i, 0))
def _degp_spec(): return pl.BlockSpec((NC, BLK, 16), lambda i: (0, i, 0))
def _full_spec(shape): return pl.BlockSpec(shape, lambda i: tuple(0 for _ in shape))


_y1_call = pl.pallas_call(
    _y1_body,
    grid=(NBLK,),
    in_specs=[_row_spec(), _full_spec((D, D)), _degp_spec()],
    out_specs=_row_spec(),
    out_shape=jax.ShapeDtypeStruct((NP, D), jnp.float32),
)

_mid_call = pl.pallas_call(
    _mid_body,
    grid=(NBLK,),
    in_specs=[pl.BlockSpec((NC, BLK, D), lambda i: (0, i, 0)), _row_spec(),
              _degp_spec(), _full_spec((D, D)), _full_spec((1, D))],
    out_specs=_row_spec(),
    out_shape=jax.ShapeDtypeStruct((NP, D), jnp.float32),
)

_fin_call = pl.pallas_call(
    _fin_body,
    grid=(NBLK,),
    in_specs=[pl.BlockSpec((NC, BLK, D), lambda i: (0, i, 0)), _row_spec(),
              _degp_spec(), _full_spec((1, D)),
              pl.BlockSpec((1, 1, BLK), lambda i: (i, 0, 0)),
              _full_spec((D, D)), _full_spec((1, D))],
    out_specs=_full_spec((G, D)),
    out_shape=jax.ShapeDtypeStruct((G, D), jnp.float32),
    scratch_shapes=[pltpu.VMEM((G, D), jnp.float32),
                    pltpu.VMEM((G, D), jnp.float32)],
)


def kernel(x, edge_index, batch, W1, b1, W2, b2, Wc, bc):
    x_p = jnp.pad(x, ((0, NP - N), (0, 0)))
    pad_src = jnp.arange(EP - E, dtype=jnp.int32) % N
    src = jnp.concatenate([edge_index[0], pad_src])
    dst = jnp.pad(edge_index[1], (0, EP - E), constant_values=NP - 1)
    srcw = src.reshape(NW, CPT, CHUNK)
    dstw = dst.reshape(NW, CPT, CHUNK)
    dstg = dst.reshape(NW, GRP, 16)
    batch3 = jnp.pad(batch, (0, NP - N), constant_values=G).reshape(NBLK, 1, BLK)
    wc_p = jnp.pad(Wc, ((0, 0), (0, D - C)))
    bc_p = jnp.pad(bc, (0, D - C)).reshape(1, D)

    degp = _deg_kernel(dstg)
    y1 = _y1_call(x_p, W1, degp)
    acc1 = _scatter_kernel(y1, srcw, dstw)
    y2 = _mid_call(acc1, y1, degp, W2, b1.reshape(1, D))
    acc2 = _scatter_kernel(y2, srcw, dstw)
    outp = _fin_call(acc2, y2, degp, b2.reshape(1, D), batch3, wc_p, bc_p)
    return outp[:, :C]
